# baseline (speedup 1.0000x reference)
import jax
import jax.numpy as jnp
from jax import lax
from jax.experimental import pallas as pl
from jax.experimental.pallas import tpu as pltpu, tpu_sc as plsc

B = 16384
F = 26
V = 100000

NC = 2
NS = 16
L = 16
NW = NC * NS
BPW = B // NW
CHUNK = 128
NCHUNK = BPW // CHUNK


def _body(x_hbm, table_hbm, out_hbm, xv, idxv, valsv, outv, sem):
    wid = lax.axis_index("s") * NC + lax.axis_index("c")
    base = wid * BPW

    pltpu.sync_copy(x_hbm.at[pl.ds(base * F, BPW * F)], xv)

    lane_f = lax.iota(jnp.int32, L) * F

    def build_fire_f(f, _):
        fbase = jnp.full((L,), f * V, jnp.int32)
        for c in range(BPW // L):
            pos = lane_f + (c * L * F + f)
            vals = plsc.load_gather(xv, [pos])
            j = c // (CHUNK // L)
            o = (c % (CHUNK // L)) * L
            idxv[f, j, pl.ds(o, L)] = vals + fbase
        for j in range(NCHUNK):
            pltpu.async_copy(
                table_hbm.at[idxv.at[f, j]],
                valsv.at[f, pl.ds(j * CHUNK, CHUNK)],
                sem,
            )
        return 0

    lax.fori_loop(0, F, build_fire_f, 0)

    def drain_f(f, _):
        for j in range(NCHUNK):
            pltpu.make_async_copy(
                table_hbm.at[idxv.at[f, j]],
                valsv.at[f, pl.ds(j * CHUNK, CHUNK)],
                sem,
            ).wait()
        return 0

    lax.fori_loop(0, F, drain_f, 0)

    def red_c(c, _):
        acc = jnp.zeros((L,), jnp.float32)
        for f in range(F):
            acc = acc + valsv[f, pl.ds(c * L, L)]
        outv[pl.ds(c * L, L)] = acc
        return 0

    lax.fori_loop(0, BPW // L, red_c, 0)

    pltpu.sync_copy(outv, out_hbm.at[pl.ds(base, BPW)])


@jax.jit
def _linear_logit(x, table_flat):
    mesh = plsc.VectorSubcoreMesh(core_axis_name="c", subcore_axis_name="s")
    return pl.kernel(
        _body,
        out_type=jax.ShapeDtypeStruct((B,), jnp.float32),
        mesh=mesh,
        compiler_params=pltpu.CompilerParams(needs_layout_passes=False),
        scratch_types=[
            pltpu.VMEM((BPW * F,), jnp.int32),
            pltpu.VMEM((F, NCHUNK, CHUNK), jnp.int32),
            pltpu.VMEM((F, BPW), jnp.float32),
            pltpu.VMEM((BPW,), jnp.float32),
            pltpu.SemaphoreType.DMA,
        ],
    )(x, table_flat)


def kernel(X, tables):
    x = X.astype(jnp.int32).reshape(B * F)
    table_flat = tables.reshape(F * V)
    return _linear_logit(x, table_flat)
